# baseline (speedup 1.0000x reference)
import functools

import jax
import jax.numpy as jnp
from jax import lax
from jax.experimental import pallas as pl
from jax.experimental.pallas import tpu as pltpu
from jax.experimental.pallas import tpu_sc as plsc

NN = 50000
D = 64
HD = 32
E = 800000
B = 4096
NNEG = 40
REG = 1e-4

NP = 50176
EP = 819200
ER = EP // 128
DUMMY = NN

NC, NS = 2, 16
RPT = NP // NS
CR = 112
KB = 2
EPTR = ER // NS
NIT = EPTR // KB

GI = 2 * B + B * NNEG
GRP = 1408
GIP = GRP * 128
GPT = GRP // NS
GB = 8

_mesh = lambda: plsc.VectorSubcoreMesh(core_axis_name="c", subcore_axis_name="s")
_sc_params = pltpu.CompilerParams(use_tc_tiling_on_sc=False)


def _deg_body(col2d, deg0, deg1, deg_sp, onesbuf, idxbuf, zbuf):
    c = lax.axis_index("c")
    s = lax.axis_index("s")
    lane = lax.iota(jnp.int32, 16)
    onev = jnp.where(lane == 0, 1.0, 0.0).astype(jnp.float32)
    zv = jnp.zeros((16,), jnp.float32)

    @pl.loop(0, 128)
    def _(r):
        onesbuf[r] = onev

    @pl.loop(0, CR)
    def _(r):
        zbuf[r] = zv

    @pl.loop(0, RPT // CR)
    def _(k):
        pltpu.sync_copy(zbuf, deg_sp.at[pl.ds(s * RPT + k * CR, CR)])

    plsc.subcore_barrier()

    base = (c * NS + s) * (ER // 32)

    @pl.loop(0, 5)
    def _(kc):
        pltpu.sync_copy(col2d.at[pl.ds(base + kc * 40, 40)], idxbuf)

        @pl.loop(0, 40)
        def _(j):
            pltpu.sync_copy(onesbuf, deg_sp.at[idxbuf.at[j]], add=True)

    plsc.subcore_barrier()

    @pl.when(c == 0)
    def _():
        @pl.loop(0, RPT // CR)
        def _(k):
            r0 = s * RPT + k * CR
            pltpu.sync_copy(deg_sp.at[pl.ds(r0, CR)], deg0.at[pl.ds(r0, CR)])

    @pl.when(c == 1)
    def _():
        @pl.loop(0, RPT // CR)
        def _(k):
            r0 = s * RPT + k * CR
            pltpu.sync_copy(deg_sp.at[pl.ds(r0, CR)], deg1.at[pl.ds(r0, CR)])


_deg_kernel = pl.kernel(
    _deg_body,
    out_type=[jax.ShapeDtypeStruct((NP, 16), jnp.float32)] * 2,
    mesh=_mesh(),
    compiler_params=_sc_params,
    scratch_types=[
        pltpu.VMEM_SHARED((NP, 16), jnp.float32),
        pltpu.VMEM((128, 16), jnp.float32),
        pltpu.VMEM((40, 128), jnp.int32),
        pltpu.VMEM((CR, 16), jnp.float32),
    ],
)


BN2 = 3136


def _prep_body(dp0, dp1, x0, dis, ylo, yhi):
    i = pl.program_id(0)
    deg = dp0[...][:, 0] + dp1[...][:, 0]
    d = jnp.where(deg > 0, lax.rsqrt(deg), 0.0)
    dis[pl.ds(pl.multiple_of(i * BN2, 128), BN2)] = d
    y = x0[...] * d[:, None]
    ylo[...] = y[:, :HD]
    yhi[...] = y[:, HD:]


_prep = pl.pallas_call(
    _prep_body,
    grid=(NP // BN2,),
    in_specs=[
        pl.BlockSpec((BN2, 16), lambda i: (i, 0)),
        pl.BlockSpec((BN2, 16), lambda i: (i, 0)),
        pl.BlockSpec((BN2, D), lambda i: (i, 0)),
    ],
    out_specs=[
        pl.BlockSpec((NP,), lambda i: (0,)),
        pl.BlockSpec((BN2, HD), lambda i: (i, 0)),
        pl.BlockSpec((BN2, HD), lambda i: (i, 0)),
    ],
    out_shape=[
        jax.ShapeDtypeStruct((NP,), jnp.float32),
        jax.ShapeDtypeStruct((NP, HD), jnp.float32),
        jax.ShapeDtypeStruct((NP, HD), jnp.float32),
    ],
)


def _layer_body(y_lo, y_hi, row2d, col2d, a_lo, a_hi,
                acc, rbuf, ridx, cidx, gsem0, gsem1, ssem0, ssem1):
    c = lax.axis_index("c")
    s = lax.axis_index("s")
    gsems = (gsem0, gsem1)
    ssems = (ssem0, ssem1)

    def half(y, aout):
        @pl.loop(0, RPT // CR)
        def _(k):
            r0 = s * RPT + k * CR
            pltpu.sync_copy(y.at[pl.ds(r0, CR)], acc.at[pl.ds(r0, CR)])

        plsc.subcore_barrier()

        base = s * EPTR

        def copy_idx(i, b):
            pltpu.sync_copy(row2d.at[pl.ds(base + i * KB, KB)], ridx.at[b])
            pltpu.sync_copy(col2d.at[pl.ds(base + i * KB, KB)], cidx.at[b])

        def fire_gathers(b):
            for j in range(KB):
                pltpu.async_copy(y.at[ridx.at[b, j]], rbuf.at[b, j], gsems[b])

        def fire_scatters(b):
            for j in range(KB):
                pltpu.async_copy(rbuf.at[b, j], acc.at[cidx.at[b, j]],
                                 ssems[b], add=True)

        def wait_gathers(b):
            for j in range(KB):
                pltpu.make_async_copy(
                    y.at[ridx.at[b, j]], rbuf.at[b, j], gsems[b]).wait()

        def wait_scatters(b):
            for j in range(KB):
                pltpu.make_async_copy(
                    rbuf.at[b, j], acc.at[cidx.at[b, j]], ssems[b]).wait()

        copy_idx(0, 0)
        fire_gathers(0)

        @pl.loop(0, NIT, step=2)
        def _(it):
            for b in range(2):
                i = it + b
                wait_gathers(b)

                @pl.when(i + 1 < NIT)
                def _():
                    @pl.when(i >= 1)
                    def _():
                        wait_scatters(1 - b)

                    copy_idx(i + 1, 1 - b)
                    fire_gathers(1 - b)

                fire_scatters(b)

        wait_scatters(0)
        wait_scatters(1)

        plsc.subcore_barrier()

        @pl.loop(0, RPT // CR)
        def _(k):
            r0 = s * RPT + k * CR
            pltpu.sync_copy(acc.at[pl.ds(r0, CR)], aout.at[pl.ds(r0, CR)])

    @pl.when(c == 0)
    def _():
        half(y_lo, a_lo)

    @pl.when(c == 1)
    def _():
        half(y_hi, a_hi)


_layer = pl.kernel(
    _layer_body,
    out_type=[jax.ShapeDtypeStruct((NP, HD), jnp.float32)] * 2,
    mesh=_mesh(),
    compiler_params=_sc_params,
    scratch_types=[
        pltpu.VMEM_SHARED((NP, HD), jnp.float32),
        pltpu.VMEM((2, KB, 128, HD), jnp.float32),
        pltpu.VMEM((2, KB, 128), jnp.int32),
        pltpu.VMEM((2, KB, 128), jnp.int32),
        pltpu.SemaphoreType.DMA,
        pltpu.SemaphoreType.DMA,
        pltpu.SemaphoreType.DMA,
        pltpu.SemaphoreType.DMA,
    ],
)


def _post1_body(alo, ahi, dis_in, zlo_in, zhi_in,
                ynlo, ynhi, zlo_out, zhi_out):
    d = dis_in[...]
    xl = d * alo[...]
    xh = d * ahi[...]
    zlo_out[...] = zlo_in[...] + xl
    zhi_out[...] = zhi_in[...] + xh
    ynlo[...] = d * xl
    ynhi[...] = d * xh


def _post2_body(alo, ahi, dis_in, zlo_in, zhi_in, zlo_out, zhi_out):
    third = 1.0 / 3.0
    d = dis_in[...]
    zlo_out[...] = (zlo_in[...] + d * alo[...]) * third
    zhi_out[...] = (zhi_in[...] + d * ahi[...]) * third


def _make_post(body, n_out):
    return pl.pallas_call(
        body,
        grid=(NP // BN2,),
        in_specs=[
            pl.BlockSpec((BN2, HD), lambda i: (i, 0)),
            pl.BlockSpec((BN2, HD), lambda i: (i, 0)),
            pl.BlockSpec((BN2, 1), lambda i: (i, 0)),
            pl.BlockSpec((BN2, HD), lambda i: (i, 0)),
            pl.BlockSpec((BN2, HD), lambda i: (i, 0)),
        ],
        out_specs=[pl.BlockSpec((BN2, HD), lambda i: (i, 0))] * n_out,
        out_shape=[jax.ShapeDtypeStruct((NP, HD), jnp.float32)] * n_out,
    )


_post1 = _make_post(_post1_body, 4)
_post2 = _make_post(_post2_body, 2)


def _gather_body(z_lo, z_hi, idx2d, out_lo, out_hi, gidx, gbuf, sem):
    c = lax.axis_index("c")
    s = lax.axis_index("s")

    def half(z, out):
        base = s * GPT
        pltpu.sync_copy(idx2d.at[pl.ds(base, GPT)], gidx)

        @pl.loop(0, GPT // GB)
        def _(it):
            descs = [
                pltpu.async_copy(z.at[gidx.at[it * GB + j]],
                                 gbuf.at[pl.ds(j * 128, 128)], sem)
                for j in range(GB)
            ]
            for dsc in descs:
                dsc.wait()
            pltpu.sync_copy(gbuf, out.at[pl.ds((base + it * GB) * 128, GB * 128)])

    @pl.when(c == 0)
    def _():
        half(z_lo, out_lo)

    @pl.when(c == 1)
    def _():
        half(z_hi, out_hi)


_gather = pl.kernel(
    _gather_body,
    out_type=[jax.ShapeDtypeStruct((GIP, HD), jnp.float32)] * 2,
    mesh=_mesh(),
    compiler_params=_sc_params,
    scratch_types=[
        pltpu.VMEM((GPT, 128), jnp.int32),
        pltpu.VMEM((GB * 128, HD), jnp.float32),
        pltpu.SemaphoreType.DMA,
    ],
)


BS = 512


def _loss_body(ulo, uhi, vlo, vhi, nlo, nhi, w, out):
    i = pl.program_id(0)
    ul = ulo[...]
    uh = uhi[...]
    pos = jnp.sum(ul * vlo[...], axis=1) + jnp.sum(uh * vhi[...], axis=1)
    nl = nlo[...]
    nh = nhi[...]
    neg = jnp.sum(nl * ul[:, None, :], axis=2) + jnp.sum(nh * uh[:, None, :], axis=2)
    coef = -jnp.sign(w[...]) + 2.0
    t = coef[:, None] * pos[:, None] - neg
    ls = jnp.minimum(t, 0.0) - jnp.log1p(jnp.exp(-jnp.abs(t)))
    reg = (jnp.sum(ul * ul) + jnp.sum(uh * uh)
           + jnp.sum(vlo[...] ** 2) + jnp.sum(vhi[...] ** 2)
           + jnp.sum(nl * nl) + jnp.sum(nh * nh))
    val = -jnp.sum(ls) + REG * reg

    @pl.when(i == 0)
    def _():
        out[0, 0] = 0.0

    out[0, 0] += val


_loss = pl.pallas_call(
    _loss_body,
    grid=(B // BS,),
    in_specs=[
        pl.BlockSpec((BS, HD), lambda i: (i, 0)),
        pl.BlockSpec((BS, HD), lambda i: (i, 0)),
        pl.BlockSpec((BS, HD), lambda i: (i, 0)),
        pl.BlockSpec((BS, HD), lambda i: (i, 0)),
        pl.BlockSpec((BS, NNEG, HD), lambda i: (i, 0, 0)),
        pl.BlockSpec((BS, NNEG, HD), lambda i: (i, 0, 0)),
        pl.BlockSpec((BS,), lambda i: (i,)),
    ],
    out_specs=pl.BlockSpec((1, 1), lambda i: (0, 0), memory_space=pltpu.SMEM),
    out_shape=jax.ShapeDtypeStruct((1, 1), jnp.float32),
)


def kernel(E_pos, E_item, u, v, w, n, edge_index):
    x0 = jnp.concatenate([E_pos, E_item], axis=0)
    x0p = jnp.pad(x0, ((0, NP - NN), (0, 0)))
    pad = jnp.full((EP - E,), DUMMY, jnp.int32)
    row2d = jnp.concatenate([edge_index[0], pad]).reshape(ER, 128)
    col2d = jnp.concatenate([edge_index[1], pad]).reshape(ER, 128)

    deg0, deg1 = _deg_kernel(col2d)
    dis, y_lo, y_hi = _prep(deg0, deg1, x0p)

    dis2d = dis[:, None]
    zin_lo = x0p[:, :HD]
    zin_hi = x0p[:, HD:]
    a1lo, a1hi = _layer(y_lo, y_hi, row2d, col2d)
    y1lo, y1hi, z1lo, z1hi = _post1(a1lo, a1hi, dis2d, zin_lo, zin_hi)
    a2lo, a2hi = _layer(y1lo, y1hi, row2d, col2d)
    zlo, zhi = _post2(a2lo, a2hi, dis2d, z1lo, z1hi)

    gpad = jnp.full((GIP - GI,), DUMMY, jnp.int32)
    all_idx = jnp.concatenate([u, v, n.reshape(-1), gpad]).reshape(GRP, 128)
    rows_lo, rows_hi = _gather(zlo, zhi, all_idx)

    ulo, vlo_r, nlo = rows_lo[:B], rows_lo[B:2 * B], rows_lo[2 * B:GI]
    uhi, vhi_r, nhi = rows_hi[:B], rows_hi[B:2 * B], rows_hi[2 * B:GI]
    out = _loss(ulo, uhi, vlo_r, vhi_r,
                nlo.reshape(B, NNEG, HD), nhi.reshape(B, NNEG, HD), w)
    return out[0, 0]


# speedup vs baseline: 1.2198x; 1.2198x over previous
import functools

import jax
import jax.numpy as jnp
from jax import lax
from jax.experimental import pallas as pl
from jax.experimental.pallas import tpu as pltpu
from jax.experimental.pallas import tpu_sc as plsc

NN = 50000
D = 64
HD = 32
E = 800000
B = 4096
NNEG = 40
REG = 1e-4

NP = 50176
EP = 811008
ER = EP // 128
DUMMY = NN

NC, NS = 2, 16
RPT = NP // NS
CR = 112
KB = 3
EPTR = ER // NS
NIT = EPTR // KB

GI = 2 * B + B * NNEG
GRP = 1408
GIP = GRP * 128
GPT = GRP // NS
GB = 8

_mesh = lambda: plsc.VectorSubcoreMesh(core_axis_name="c", subcore_axis_name="s")
_sc_params = pltpu.CompilerParams(use_tc_tiling_on_sc=False)


def _deg_body(col2d, deg0, deg1, deg_sp, onesbuf, idxbuf, zbuf):
    c = lax.axis_index("c")
    s = lax.axis_index("s")
    lane = lax.iota(jnp.int32, 16)
    onev = jnp.where(lane == 0, 1.0, 0.0).astype(jnp.float32)
    zv = jnp.zeros((16,), jnp.float32)

    @pl.loop(0, 128)
    def _(r):
        onesbuf[r] = onev

    @pl.loop(0, CR)
    def _(r):
        zbuf[r] = zv

    @pl.loop(0, RPT // CR)
    def _(k):
        pltpu.sync_copy(zbuf, deg_sp.at[pl.ds(s * RPT + k * CR, CR)])

    plsc.subcore_barrier()

    base = (c * NS + s) * (ER // 32)

    @pl.loop(0, 6)
    def _(kc):
        pltpu.sync_copy(col2d.at[pl.ds(base + kc * 33, 33)], idxbuf)

        @pl.loop(0, 33)
        def _(j):
            pltpu.sync_copy(onesbuf, deg_sp.at[idxbuf.at[j]], add=True)

    plsc.subcore_barrier()

    @pl.when(c == 0)
    def _():
        @pl.loop(0, RPT // CR)
        def _(k):
            r0 = s * RPT + k * CR
            pltpu.sync_copy(deg_sp.at[pl.ds(r0, CR)], deg0.at[pl.ds(r0, CR)])

    @pl.when(c == 1)
    def _():
        @pl.loop(0, RPT // CR)
        def _(k):
            r0 = s * RPT + k * CR
            pltpu.sync_copy(deg_sp.at[pl.ds(r0, CR)], deg1.at[pl.ds(r0, CR)])


_deg_kernel = pl.kernel(
    _deg_body,
    out_type=[jax.ShapeDtypeStruct((NP, 16), jnp.float32)] * 2,
    mesh=_mesh(),
    compiler_params=_sc_params,
    scratch_types=[
        pltpu.VMEM_SHARED((NP, 16), jnp.float32),
        pltpu.VMEM((128, 16), jnp.float32),
        pltpu.VMEM((33, 128), jnp.int32),
        pltpu.VMEM((CR, 16), jnp.float32),
    ],
)


BN2 = 3136


def _prep_body(dp0, dp1, x0, dis, ylo, yhi):
    i = pl.program_id(0)
    deg = dp0[...][:, 0] + dp1[...][:, 0]
    d = jnp.where(deg > 0, lax.rsqrt(deg), 0.0)
    dis[pl.ds(pl.multiple_of(i * BN2, 128), BN2)] = d
    y = x0[...] * d[:, None]
    ylo[...] = y[:, :HD]
    yhi[...] = y[:, HD:]


_prep = pl.pallas_call(
    _prep_body,
    grid=(NP // BN2,),
    in_specs=[
        pl.BlockSpec((BN2, 16), lambda i: (i, 0)),
        pl.BlockSpec((BN2, 16), lambda i: (i, 0)),
        pl.BlockSpec((BN2, D), lambda i: (i, 0)),
    ],
    out_specs=[
        pl.BlockSpec((NP,), lambda i: (0,)),
        pl.BlockSpec((BN2, HD), lambda i: (i, 0)),
        pl.BlockSpec((BN2, HD), lambda i: (i, 0)),
    ],
    out_shape=[
        jax.ShapeDtypeStruct((NP,), jnp.float32),
        jax.ShapeDtypeStruct((NP, HD), jnp.float32),
        jax.ShapeDtypeStruct((NP, HD), jnp.float32),
    ],
)


def _layer_body(y_lo, y_hi, row1d, col1d, a_lo, a_hi,
                acc, rbuf, ridx, cidx, gsem0, gsem1, ssem0, ssem1):
    c = lax.axis_index("c")
    s = lax.axis_index("s")
    gsems = (gsem0, gsem1)
    ssems = (ssem0, ssem1)

    def half(y, aout):
        @pl.loop(0, RPT // CR)
        def _(k):
            r0 = s * RPT + k * CR
            pltpu.sync_copy(y.at[pl.ds(r0, CR)], acc.at[pl.ds(r0, CR)])

        plsc.subcore_barrier()

        base = s * EPTR

        def copy_idx(i, b):
            e0 = (base + i * KB) * 128
            pltpu.sync_copy(row1d.at[pl.ds(e0, KB * 128)], ridx.at[b])
            pltpu.sync_copy(col1d.at[pl.ds(e0, KB * 128)], cidx.at[b])

        def fire_gathers(b):
            pltpu.async_copy(y.at[ridx.at[b]], rbuf.at[b], gsems[b])

        def fire_scatters(b):
            pltpu.async_copy(rbuf.at[b], acc.at[cidx.at[b]], ssems[b], add=True)

        def wait_gathers(b):
            pltpu.make_async_copy(y.at[ridx.at[b]], rbuf.at[b], gsems[b]).wait()

        def wait_scatters(b):
            pltpu.make_async_copy(
                rbuf.at[b], acc.at[cidx.at[b]], ssems[b]).wait()

        copy_idx(0, 0)
        fire_gathers(0)

        @pl.loop(0, NIT, step=2)
        def _(it):
            for b in range(2):
                i = it + b
                wait_gathers(b)

                @pl.when(i + 1 < NIT)
                def _():
                    @pl.when(i >= 1)
                    def _():
                        wait_scatters(1 - b)

                    copy_idx(i + 1, 1 - b)
                    fire_gathers(1 - b)

                fire_scatters(b)

        wait_scatters(0)
        wait_scatters(1)

        plsc.subcore_barrier()

        @pl.loop(0, RPT // CR)
        def _(k):
            r0 = s * RPT + k * CR
            pltpu.sync_copy(acc.at[pl.ds(r0, CR)], aout.at[pl.ds(r0, CR)])

    @pl.when(c == 0)
    def _():
        half(y_lo, a_lo)

    @pl.when(c == 1)
    def _():
        half(y_hi, a_hi)


_layer = pl.kernel(
    _layer_body,
    out_type=[jax.ShapeDtypeStruct((NP, HD), jnp.float32)] * 2,
    mesh=_mesh(),
    compiler_params=_sc_params,
    scratch_types=[
        pltpu.VMEM_SHARED((NP, HD), jnp.float32),
        pltpu.VMEM((2, KB * 128, HD), jnp.float32),
        pltpu.VMEM((2, KB * 128), jnp.int32),
        pltpu.VMEM((2, KB * 128), jnp.int32),
        pltpu.SemaphoreType.DMA,
        pltpu.SemaphoreType.DMA,
        pltpu.SemaphoreType.DMA,
        pltpu.SemaphoreType.DMA,
    ],
)


def _post1_body(alo, ahi, dis_in, zlo_in, zhi_in,
                ynlo, ynhi, zlo_out, zhi_out):
    d = dis_in[...]
    xl = d * alo[...]
    xh = d * ahi[...]
    zlo_out[...] = zlo_in[...] + xl
    zhi_out[...] = zhi_in[...] + xh
    ynlo[...] = d * xl
    ynhi[...] = d * xh


def _post2_body(alo, ahi, dis_in, zlo_in, zhi_in, zlo_out, zhi_out):
    third = 1.0 / 3.0
    d = dis_in[...]
    zlo_out[...] = (zlo_in[...] + d * alo[...]) * third
    zhi_out[...] = (zhi_in[...] + d * ahi[...]) * third


def _make_post(body, n_out):
    return pl.pallas_call(
        body,
        grid=(NP // BN2,),
        in_specs=[
            pl.BlockSpec((BN2, HD), lambda i: (i, 0)),
            pl.BlockSpec((BN2, HD), lambda i: (i, 0)),
            pl.BlockSpec((BN2, 1), lambda i: (i, 0)),
            pl.BlockSpec((BN2, HD), lambda i: (i, 0)),
            pl.BlockSpec((BN2, HD), lambda i: (i, 0)),
        ],
        out_specs=[pl.BlockSpec((BN2, HD), lambda i: (i, 0))] * n_out,
        out_shape=[jax.ShapeDtypeStruct((NP, HD), jnp.float32)] * n_out,
    )


_post1 = _make_post(_post1_body, 4)
_post2 = _make_post(_post2_body, 2)


def _gather_body(z_lo, z_hi, idx2d, out_lo, out_hi, gidx, gbuf, sem):
    c = lax.axis_index("c")
    s = lax.axis_index("s")

    def half(z, out):
        base = s * GPT
        pltpu.sync_copy(idx2d.at[pl.ds(base, GPT)], gidx)

        @pl.loop(0, GPT // GB)
        def _(it):
            descs = [
                pltpu.async_copy(z.at[gidx.at[it * GB + j]],
                                 gbuf.at[pl.ds(j * 128, 128)], sem)
                for j in range(GB)
            ]
            for dsc in descs:
                dsc.wait()
            pltpu.sync_copy(gbuf, out.at[pl.ds((base + it * GB) * 128, GB * 128)])

    @pl.when(c == 0)
    def _():
        half(z_lo, out_lo)

    @pl.when(c == 1)
    def _():
        half(z_hi, out_hi)


_gather = pl.kernel(
    _gather_body,
    out_type=[jax.ShapeDtypeStruct((GIP, HD), jnp.float32)] * 2,
    mesh=_mesh(),
    compiler_params=_sc_params,
    scratch_types=[
        pltpu.VMEM((GPT, 128), jnp.int32),
        pltpu.VMEM((GB * 128, HD), jnp.float32),
        pltpu.SemaphoreType.DMA,
    ],
)


BS = 512


def _loss_body(ulo, uhi, vlo, vhi, nlo, nhi, w, out):
    i = pl.program_id(0)
    ul = ulo[...]
    uh = uhi[...]
    pos = jnp.sum(ul * vlo[...], axis=1) + jnp.sum(uh * vhi[...], axis=1)
    nl = nlo[...]
    nh = nhi[...]
    neg = jnp.sum(nl * ul[:, None, :], axis=2) + jnp.sum(nh * uh[:, None, :], axis=2)
    coef = -jnp.sign(w[...]) + 2.0
    t = coef[:, None] * pos[:, None] - neg
    ls = jnp.minimum(t, 0.0) - jnp.log1p(jnp.exp(-jnp.abs(t)))
    reg = (jnp.sum(ul * ul) + jnp.sum(uh * uh)
           + jnp.sum(vlo[...] ** 2) + jnp.sum(vhi[...] ** 2)
           + jnp.sum(nl * nl) + jnp.sum(nh * nh))
    val = -jnp.sum(ls) + REG * reg

    @pl.when(i == 0)
    def _():
        out[0, 0] = 0.0

    out[0, 0] += val


_loss = pl.pallas_call(
    _loss_body,
    grid=(B // BS,),
    in_specs=[
        pl.BlockSpec((BS, HD), lambda i: (i, 0)),
        pl.BlockSpec((BS, HD), lambda i: (i, 0)),
        pl.BlockSpec((BS, HD), lambda i: (i, 0)),
        pl.BlockSpec((BS, HD), lambda i: (i, 0)),
        pl.BlockSpec((BS, NNEG, HD), lambda i: (i, 0, 0)),
        pl.BlockSpec((BS, NNEG, HD), lambda i: (i, 0, 0)),
        pl.BlockSpec((BS,), lambda i: (i,)),
    ],
    out_specs=pl.BlockSpec((1, 1), lambda i: (0, 0), memory_space=pltpu.SMEM),
    out_shape=jax.ShapeDtypeStruct((1, 1), jnp.float32),
)


def kernel(E_pos, E_item, u, v, w, n, edge_index):
    x0 = jnp.concatenate([E_pos, E_item], axis=0)
    x0p = jnp.pad(x0, ((0, NP - NN), (0, 0)))
    pad = jnp.full((EP - E,), DUMMY, jnp.int32)
    row2d = jnp.concatenate([edge_index[0], pad]).reshape(ER, 128)
    col2d = jnp.concatenate([edge_index[1], pad]).reshape(ER, 128)

    deg0, deg1 = _deg_kernel(col2d)
    dis, y_lo, y_hi = _prep(deg0, deg1, x0p)

    dis2d = dis[:, None]
    zin_lo = x0p[:, :HD]
    zin_hi = x0p[:, HD:]
    row1d = row2d.reshape(EP)
    col1d = col2d.reshape(EP)
    a1lo, a1hi = _layer(y_lo, y_hi, row1d, col1d)
    y1lo, y1hi, z1lo, z1hi = _post1(a1lo, a1hi, dis2d, zin_lo, zin_hi)
    a2lo, a2hi = _layer(y1lo, y1hi, row1d, col1d)
    zlo, zhi = _post2(a2lo, a2hi, dis2d, z1lo, z1hi)

    gpad = jnp.full((GIP - GI,), DUMMY, jnp.int32)
    all_idx = jnp.concatenate([u, v, n.reshape(-1), gpad]).reshape(GRP, 128)
    rows_lo, rows_hi = _gather(zlo, zhi, all_idx)

    ulo, vlo_r, nlo = rows_lo[:B], rows_lo[B:2 * B], rows_lo[2 * B:GI]
    uhi, vhi_r, nhi = rows_hi[:B], rows_hi[B:2 * B], rows_hi[2 * B:GI]
    out = _loss(ulo, uhi, vlo_r, vhi_r,
                nlo.reshape(B, NNEG, HD), nhi.reshape(B, NNEG, HD), w)
    return out[0, 0]


# speedup vs baseline: 1.3957x; 1.1443x over previous
import functools

import jax
import jax.numpy as jnp
from jax import lax
from jax.experimental import pallas as pl
from jax.experimental.pallas import tpu as pltpu
from jax.experimental.pallas import tpu_sc as plsc

NN = 50000
D = 64
HD = 32
E = 800000
B = 4096
NNEG = 40
REG = 1e-4

NP = 50176
EP = 811008
ER = EP // 128
DUMMY = NN

NC, NS = 2, 16
RPT = NP // NS
CR = 112
KB = 3
EPTR = ER // NS
NIT = EPTR // KB

GI = 2 * B + B * NNEG
GRP = 1408
GIP = GRP * 128
GPT = GRP // NS
GB = 8

_mesh = lambda: plsc.VectorSubcoreMesh(core_axis_name="c", subcore_axis_name="s")
_sc_params = pltpu.CompilerParams(use_tc_tiling_on_sc=False)


def _deg_body(col2d, deg0, deg1, deg_sp, onesbuf, idxbuf, zbuf):
    c = lax.axis_index("c")
    s = lax.axis_index("s")
    lane = lax.iota(jnp.int32, 16)
    onev = jnp.where(lane == 0, 1.0, 0.0).astype(jnp.float32)
    zv = jnp.zeros((16,), jnp.float32)

    @pl.loop(0, 128)
    def _(r):
        onesbuf[r] = onev

    @pl.loop(0, CR)
    def _(r):
        zbuf[r] = zv

    @pl.loop(0, RPT // CR)
    def _(k):
        pltpu.sync_copy(zbuf, deg_sp.at[pl.ds(s * RPT + k * CR, CR)])

    plsc.subcore_barrier()

    base = (c * NS + s) * (ER // 32)

    @pl.loop(0, 6)
    def _(kc):
        pltpu.sync_copy(col2d.at[pl.ds(base + kc * 33, 33)], idxbuf)

        @pl.loop(0, 33)
        def _(j):
            pltpu.sync_copy(onesbuf, deg_sp.at[idxbuf.at[j]], add=True)

    plsc.subcore_barrier()

    @pl.when(c == 0)
    def _():
        @pl.loop(0, RPT // CR)
        def _(k):
            r0 = s * RPT + k * CR
            pltpu.sync_copy(deg_sp.at[pl.ds(r0, CR)], deg0.at[pl.ds(r0, CR)])

    @pl.when(c == 1)
    def _():
        @pl.loop(0, RPT // CR)
        def _(k):
            r0 = s * RPT + k * CR
            pltpu.sync_copy(deg_sp.at[pl.ds(r0, CR)], deg1.at[pl.ds(r0, CR)])


_deg_kernel = pl.kernel(
    _deg_body,
    out_type=[jax.ShapeDtypeStruct((NP, 16), jnp.float32)] * 2,
    mesh=_mesh(),
    compiler_params=_sc_params,
    scratch_types=[
        pltpu.VMEM_SHARED((NP, 16), jnp.float32),
        pltpu.VMEM((128, 16), jnp.float32),
        pltpu.VMEM((33, 128), jnp.int32),
        pltpu.VMEM((CR, 16), jnp.float32),
    ],
)


BN2 = 3136


def _prep_body(dp0, dp1, x0, dis, ylo, yhi):
    i = pl.program_id(0)
    deg = dp0[...][:, 0] + dp1[...][:, 0]
    d = jnp.where(deg > 0, lax.rsqrt(deg), 0.0)
    dis[pl.ds(pl.multiple_of(i * BN2, 128), BN2)] = d
    y = x0[...] * d[:, None]
    ylo[...] = y[:, :HD]
    yhi[...] = y[:, HD:]


_prep = pl.pallas_call(
    _prep_body,
    grid=(NP // BN2,),
    in_specs=[
        pl.BlockSpec((BN2, 16), lambda i: (i, 0)),
        pl.BlockSpec((BN2, 16), lambda i: (i, 0)),
        pl.BlockSpec((BN2, D), lambda i: (i, 0)),
    ],
    out_specs=[
        pl.BlockSpec((NP,), lambda i: (0,)),
        pl.BlockSpec((BN2, HD), lambda i: (i, 0)),
        pl.BlockSpec((BN2, HD), lambda i: (i, 0)),
    ],
    out_shape=[
        jax.ShapeDtypeStruct((NP,), jnp.float32),
        jax.ShapeDtypeStruct((NP, HD), jnp.float32),
        jax.ShapeDtypeStruct((NP, HD), jnp.float32),
    ],
)


def _layer_body(y_lo, y_hi, row1d, col1d, a_lo, a_hi,
                acc, rbuf, ridx, cidx,
                gsem0, gsem1, ssem0, ssem1, isem0, isem1):
    c = lax.axis_index("c")
    s = lax.axis_index("s")
    gsems = (gsem0, gsem1)
    ssems = (ssem0, ssem1)
    isems = (isem0, isem1)

    def half(y, aout):
        @pl.loop(0, RPT // CR)
        def _(k):
            r0 = s * RPT + k * CR
            pltpu.sync_copy(y.at[pl.ds(r0, CR)], acc.at[pl.ds(r0, CR)])

        plsc.subcore_barrier()

        base = s * EPTR

        def copy_idx(i, ib, sem):
            e0 = (base + i * KB) * 128
            pltpu.async_copy(row1d.at[pl.ds(e0, KB * 128)], ridx.at[ib], sem)
            pltpu.async_copy(col1d.at[pl.ds(e0, KB * 128)], cidx.at[ib], sem)

        def wait_idx(ib, sem):
            pltpu.make_async_copy(
                row1d.at[pl.ds(0, KB * 128)], ridx.at[ib], sem).wait()
            pltpu.make_async_copy(
                col1d.at[pl.ds(0, KB * 128)], cidx.at[ib], sem).wait()

        def fire_gathers(b, ib):
            pltpu.async_copy(y.at[ridx.at[ib]], rbuf.at[b], gsems[b])

        def fire_scatters(b, ib):
            pltpu.async_copy(rbuf.at[b], acc.at[cidx.at[ib]], ssems[b], add=True)

        def wait_gathers(b, ib):
            pltpu.make_async_copy(y.at[ridx.at[ib]], rbuf.at[b], gsems[b]).wait()

        def wait_scatters(b, ib):
            pltpu.make_async_copy(
                rbuf.at[b], acc.at[cidx.at[ib]], ssems[b]).wait()

        copy_idx(0, 0, isems[0])
        wait_idx(0, isems[0])
        copy_idx(1, 1, isems[1])
        fire_gathers(0, 0)

        @pl.loop(0, NIT, step=6)
        def _(it):
            for k in range(6):
                i = it + k
                b = k % 2
                ib = k % 3
                wait_gathers(b, ib)

                @pl.when(i + 1 < NIT)
                def _():
                    @pl.when(i >= 1)
                    def _():
                        wait_scatters(1 - b, (k + 2) % 3)

                    @pl.when(i + 2 < NIT)
                    def _():
                        copy_idx(i + 2, (k + 2) % 3, isems[b])

                    wait_idx((k + 1) % 3, isems[1 - b])
                    fire_gathers(1 - b, (k + 1) % 3)

                fire_scatters(b, ib)

        wait_scatters(0, (NIT - 2) % 3)
        wait_scatters(1, (NIT - 1) % 3)

        plsc.subcore_barrier()

        @pl.loop(0, RPT // CR)
        def _(k):
            r0 = s * RPT + k * CR
            pltpu.sync_copy(acc.at[pl.ds(r0, CR)], aout.at[pl.ds(r0, CR)])

    @pl.when(c == 0)
    def _():
        half(y_lo, a_lo)

    @pl.when(c == 1)
    def _():
        half(y_hi, a_hi)


_layer = pl.kernel(
    _layer_body,
    out_type=[jax.ShapeDtypeStruct((NP, HD), jnp.float32)] * 2,
    mesh=_mesh(),
    compiler_params=_sc_params,
    scratch_types=[
        pltpu.VMEM_SHARED((NP, HD), jnp.float32),
        pltpu.VMEM((2, KB * 128, HD), jnp.float32),
        pltpu.VMEM((3, KB * 128), jnp.int32),
        pltpu.VMEM((3, KB * 128), jnp.int32),
        pltpu.SemaphoreType.DMA,
        pltpu.SemaphoreType.DMA,
        pltpu.SemaphoreType.DMA,
        pltpu.SemaphoreType.DMA,
        pltpu.SemaphoreType.DMA,
        pltpu.SemaphoreType.DMA,
    ],
)


def _post1_body(alo, ahi, dis_in, zlo_in, zhi_in,
                ynlo, ynhi, zlo_out, zhi_out):
    d = dis_in[...]
    xl = d * alo[...]
    xh = d * ahi[...]
    zlo_out[...] = zlo_in[...] + xl
    zhi_out[...] = zhi_in[...] + xh
    ynlo[...] = d * xl
    ynhi[...] = d * xh


def _post2_body(alo, ahi, dis_in, zlo_in, zhi_in, zlo_out, zhi_out):
    third = 1.0 / 3.0
    d = dis_in[...]
    zlo_out[...] = (zlo_in[...] + d * alo[...]) * third
    zhi_out[...] = (zhi_in[...] + d * ahi[...]) * third


def _make_post(body, n_out):
    return pl.pallas_call(
        body,
        grid=(NP // BN2,),
        in_specs=[
            pl.BlockSpec((BN2, HD), lambda i: (i, 0)),
            pl.BlockSpec((BN2, HD), lambda i: (i, 0)),
            pl.BlockSpec((BN2, 1), lambda i: (i, 0)),
            pl.BlockSpec((BN2, HD), lambda i: (i, 0)),
            pl.BlockSpec((BN2, HD), lambda i: (i, 0)),
        ],
        out_specs=[pl.BlockSpec((BN2, HD), lambda i: (i, 0))] * n_out,
        out_shape=[jax.ShapeDtypeStruct((NP, HD), jnp.float32)] * n_out,
    )


_post1 = _make_post(_post1_body, 4)
_post2 = _make_post(_post2_body, 2)


def _gather_body(z_lo, z_hi, idx1d, out_lo, out_hi,
                 gidx, gbuf, gsem0, gsem1, osem0, osem1):
    c = lax.axis_index("c")
    s = lax.axis_index("s")
    gsems = (gsem0, gsem1)
    osems = (osem0, osem1)
    NSLAB = GPT // 8

    def half(z, out):
        base = s * GPT * 128
        pltpu.sync_copy(idx1d.at[pl.ds(base, GPT * 128)], gidx)

        def fire_g(b, it):
            pltpu.async_copy(z.at[gidx.at[pl.ds(it * 1024, 1024)]],
                             gbuf.at[b], gsems[b])

        def wait_g(b, it):
            pltpu.make_async_copy(z.at[gidx.at[pl.ds(it * 1024, 1024)]],
                                  gbuf.at[b], gsems[b]).wait()

        def fire_o(b, it):
            pltpu.async_copy(gbuf.at[b],
                             out.at[pl.ds(base + it * 1024, 1024)], osems[b])

        def wait_o(b, it):
            pltpu.make_async_copy(gbuf.at[b],
                                  out.at[pl.ds(base + it * 1024, 1024)],
                                  osems[b]).wait()

        fire_g(0, 0)
        for it in range(NSLAB):
            b = it % 2
            wait_g(b, it)
            if it + 1 < NSLAB:
                if it >= 1:
                    wait_o(1 - b, it - 1)
                fire_g(1 - b, it + 1)
            fire_o(b, it)
        wait_o(1, NSLAB - 2)
        wait_o(0, NSLAB - 1)

    @pl.when(c == 0)
    def _():
        half(z_lo, out_lo)

    @pl.when(c == 1)
    def _():
        half(z_hi, out_hi)


_gather = pl.kernel(
    _gather_body,
    out_type=[jax.ShapeDtypeStruct((GIP, HD), jnp.float32)] * 2,
    mesh=_mesh(),
    compiler_params=_sc_params,
    scratch_types=[
        pltpu.VMEM((GPT * 128,), jnp.int32),
        pltpu.VMEM((2, 1024, HD), jnp.float32),
        pltpu.SemaphoreType.DMA,
        pltpu.SemaphoreType.DMA,
        pltpu.SemaphoreType.DMA,
        pltpu.SemaphoreType.DMA,
    ],
)


BS = 512


def _loss_body(ulo, uhi, vlo, vhi, nlo, nhi, w, out):
    i = pl.program_id(0)
    ul = ulo[...]
    uh = uhi[...]
    pos = jnp.sum(ul * vlo[...], axis=1) + jnp.sum(uh * vhi[...], axis=1)
    nl = nlo[...]
    nh = nhi[...]
    neg = jnp.sum(nl * ul[:, None, :], axis=2) + jnp.sum(nh * uh[:, None, :], axis=2)
    coef = -jnp.sign(w[...]) + 2.0
    t = coef[:, None] * pos[:, None] - neg
    ls = jnp.minimum(t, 0.0) - jnp.log1p(jnp.exp(-jnp.abs(t)))
    reg = (jnp.sum(ul * ul) + jnp.sum(uh * uh)
           + jnp.sum(vlo[...] ** 2) + jnp.sum(vhi[...] ** 2)
           + jnp.sum(nl * nl) + jnp.sum(nh * nh))
    val = -jnp.sum(ls) + REG * reg

    @pl.when(i == 0)
    def _():
        out[0, 0] = 0.0

    out[0, 0] += val


_loss = pl.pallas_call(
    _loss_body,
    grid=(B // BS,),
    in_specs=[
        pl.BlockSpec((BS, HD), lambda i: (i, 0)),
        pl.BlockSpec((BS, HD), lambda i: (i, 0)),
        pl.BlockSpec((BS, HD), lambda i: (i, 0)),
        pl.BlockSpec((BS, HD), lambda i: (i, 0)),
        pl.BlockSpec((BS, NNEG, HD), lambda i: (i, 0, 0)),
        pl.BlockSpec((BS, NNEG, HD), lambda i: (i, 0, 0)),
        pl.BlockSpec((BS,), lambda i: (i,)),
    ],
    out_specs=pl.BlockSpec((1, 1), lambda i: (0, 0), memory_space=pltpu.SMEM),
    out_shape=jax.ShapeDtypeStruct((1, 1), jnp.float32),
)


def kernel(E_pos, E_item, u, v, w, n, edge_index):
    x0 = jnp.concatenate([E_pos, E_item], axis=0)
    x0p = jnp.pad(x0, ((0, NP - NN), (0, 0)))
    pad = jnp.full((EP - E,), DUMMY, jnp.int32)
    row2d = jnp.concatenate([edge_index[0], pad]).reshape(ER, 128)
    col2d = jnp.concatenate([edge_index[1], pad]).reshape(ER, 128)

    deg0, deg1 = _deg_kernel(col2d)
    dis, y_lo, y_hi = _prep(deg0, deg1, x0p)

    dis2d = dis[:, None]
    zin_lo = x0p[:, :HD]
    zin_hi = x0p[:, HD:]
    row1d = row2d.reshape(EP)
    col1d = col2d.reshape(EP)
    a1lo, a1hi = _layer(y_lo, y_hi, row1d, col1d)
    y1lo, y1hi, z1lo, z1hi = _post1(a1lo, a1hi, dis2d, zin_lo, zin_hi)
    a2lo, a2hi = _layer(y1lo, y1hi, row1d, col1d)
    zlo, zhi = _post2(a2lo, a2hi, dis2d, z1lo, z1hi)

    gpad = jnp.full((GIP - GI,), DUMMY, jnp.int32)
    all_idx = jnp.concatenate([u, v, n.reshape(-1), gpad])
    rows_lo, rows_hi = _gather(zlo, zhi, all_idx)

    ulo, vlo_r, nlo = rows_lo[:B], rows_lo[B:2 * B], rows_lo[2 * B:GI]
    uhi, vhi_r, nhi = rows_hi[:B], rows_hi[B:2 * B], rows_hi[2 * B:GI]
    out = _loss(ulo, uhi, vlo_r, vhi_r,
                nlo.reshape(B, NNEG, HD), nhi.reshape(B, NNEG, HD), w)
    return out[0, 0]


# speedup vs baseline: 1.5777x; 1.1304x over previous
import functools

import jax
import jax.numpy as jnp
from jax import lax
from jax.experimental import pallas as pl
from jax.experimental.pallas import tpu as pltpu
from jax.experimental.pallas import tpu_sc as plsc

NN = 50000
D = 64
HD = 32
E = 800000
B = 4096
NNEG = 40
REG = 1e-4

NP = 50176
EP = 811008
ER = EP // 128
DUMMY = NN

NC, NS = 2, 16
RPT = NP // NS
CR = 112
KB = 3
EPTR = ER // NS
NIT = EPTR // KB

GI = 2 * B + B * NNEG
GRP = 1408
GIP = GRP * 128
GPT = GRP // NS
GB = 8

_mesh = lambda: plsc.VectorSubcoreMesh(core_axis_name="c", subcore_axis_name="s")
_sc_params = pltpu.CompilerParams(use_tc_tiling_on_sc=False)


def _deg_body(col2d, deg0, deg1, deg_sp, onesbuf, idxbuf, zbuf):
    c = lax.axis_index("c")
    s = lax.axis_index("s")
    lane = lax.iota(jnp.int32, 16)
    onev = jnp.where(lane == 0, 1.0, 0.0).astype(jnp.float32)
    zv = jnp.zeros((16,), jnp.float32)

    @pl.loop(0, 128)
    def _(r):
        onesbuf[r] = onev

    @pl.loop(0, CR)
    def _(r):
        zbuf[r] = zv

    @pl.loop(0, RPT // CR)
    def _(k):
        pltpu.sync_copy(zbuf, deg_sp.at[pl.ds(s * RPT + k * CR, CR)])

    plsc.subcore_barrier()

    base = (c * NS + s) * (ER // 32)

    @pl.loop(0, 6)
    def _(kc):
        pltpu.sync_copy(col2d.at[pl.ds(base + kc * 33, 33)], idxbuf)

        @pl.loop(0, 33)
        def _(j):
            pltpu.sync_copy(onesbuf, deg_sp.at[idxbuf.at[j]], add=True)

    plsc.subcore_barrier()

    @pl.when(c == 0)
    def _():
        @pl.loop(0, RPT // CR)
        def _(k):
            r0 = s * RPT + k * CR
            pltpu.sync_copy(deg_sp.at[pl.ds(r0, CR)], deg0.at[pl.ds(r0, CR)])

    @pl.when(c == 1)
    def _():
        @pl.loop(0, RPT // CR)
        def _(k):
            r0 = s * RPT + k * CR
            pltpu.sync_copy(deg_sp.at[pl.ds(r0, CR)], deg1.at[pl.ds(r0, CR)])


_deg_kernel = pl.kernel(
    _deg_body,
    out_type=[jax.ShapeDtypeStruct((NP, 16), jnp.float32)] * 2,
    mesh=_mesh(),
    compiler_params=_sc_params,
    scratch_types=[
        pltpu.VMEM_SHARED((NP, 16), jnp.float32),
        pltpu.VMEM((128, 16), jnp.float32),
        pltpu.VMEM((33, 128), jnp.int32),
        pltpu.VMEM((CR, 16), jnp.float32),
    ],
)


BN2 = 3136


def _prep_body(dp0, dp1, x0, dis, ylo, yhi):
    i = pl.program_id(0)
    deg = dp0[...][:, 0] + dp1[...][:, 0]
    d = jnp.where(deg > 0, lax.rsqrt(deg), 0.0)
    dis[pl.ds(pl.multiple_of(i * BN2, 128), BN2)] = d
    y = x0[...] * d[:, None]
    ylo[...] = y[:, :HD]
    yhi[...] = y[:, HD:]


_prep = pl.pallas_call(
    _prep_body,
    grid=(NP // BN2,),
    in_specs=[
        pl.BlockSpec((BN2, 16), lambda i: (i, 0)),
        pl.BlockSpec((BN2, 16), lambda i: (i, 0)),
        pl.BlockSpec((BN2, D), lambda i: (i, 0)),
    ],
    out_specs=[
        pl.BlockSpec((NP,), lambda i: (0,)),
        pl.BlockSpec((BN2, HD), lambda i: (i, 0)),
        pl.BlockSpec((BN2, HD), lambda i: (i, 0)),
    ],
    out_shape=[
        jax.ShapeDtypeStruct((NP,), jnp.float32),
        jax.ShapeDtypeStruct((NP, HD), jnp.float32),
        jax.ShapeDtypeStruct((NP, HD), jnp.float32),
    ],
)


CRE = 112


def _layers_body(y_lo, y_hi, row1d, col1d, dis, x0_lo, x0_hi,
                 y1_lo, y1_hi, z_lo, z_hi,
                 acc, rbuf, ridx, cidx, dbuf,
                 gsem0, gsem1, ssem0, ssem1, isem0, isem1):
    c = lax.axis_index("c")
    s = lax.axis_index("s")
    gsems = (gsem0, gsem1)
    ssems = (ssem0, ssem1)
    isems = (isem0, isem1)

    def edge_pass(y):
        base = s * EPTR

        def copy_idx(i, ib, sem):
            e0 = (base + i * KB) * 128
            pltpu.async_copy(row1d.at[pl.ds(e0, KB * 128)], ridx.at[ib], sem)
            pltpu.async_copy(col1d.at[pl.ds(e0, KB * 128)], cidx.at[ib], sem)

        def wait_idx(ib, sem):
            pltpu.make_async_copy(
                row1d.at[pl.ds(0, KB * 128)], ridx.at[ib], sem).wait()
            pltpu.make_async_copy(
                col1d.at[pl.ds(0, KB * 128)], cidx.at[ib], sem).wait()

        def fire_gathers(b, ib):
            pltpu.async_copy(y.at[ridx.at[ib]], rbuf.at[b], gsems[b])

        def fire_scatters(b, ib):
            pltpu.async_copy(rbuf.at[b], acc.at[cidx.at[ib]], ssems[b],
                             add=True)

        def wait_gathers(b, ib):
            pltpu.make_async_copy(y.at[ridx.at[ib]], rbuf.at[b],
                                  gsems[b]).wait()

        def wait_scatters(b, ib):
            pltpu.make_async_copy(
                rbuf.at[b], acc.at[cidx.at[ib]], ssems[b]).wait()

        copy_idx(0, 0, isems[0])
        wait_idx(0, isems[0])
        copy_idx(1, 1, isems[1])
        fire_gathers(0, 0)

        @pl.loop(0, NIT, step=6)
        def _(it):
            for k in range(6):
                i = it + k
                b = k % 2
                ib = k % 3
                wait_gathers(b, ib)

                @pl.when(i + 1 < NIT)
                def _():
                    @pl.when(i >= 1)
                    def _():
                        wait_scatters(1 - b, (k + 2) % 3)

                    @pl.when(i + 2 < NIT)
                    def _():
                        copy_idx(i + 2, (k + 2) % 3, isems[b])

                    wait_idx((k + 1) % 3, isems[1 - b])
                    fire_gathers(1 - b, (k + 1) % 3)

                fire_scatters(b, ib)

        wait_scatters(0, (NIT - 2) % 3)
        wait_scatters(1, (NIT - 1) % 3)

    def half(y0, x0h, y1t, zo):
        st0 = rbuf.at[0, pl.ds(0, CRE)]
        st1 = rbuf.at[1, pl.ds(0, CRE)]

        @pl.loop(0, RPT // CR)
        def _(k):
            r0 = s * RPT + k * CR
            pltpu.sync_copy(y0.at[pl.ds(r0, CR)], acc.at[pl.ds(r0, CR)])

        plsc.subcore_barrier()
        edge_pass(y0)
        plsc.subcore_barrier()

        @pl.loop(0, RPT // CRE)
        def _(kc):
            r0 = s * RPT + kc * CRE
            pltpu.sync_copy(acc.at[pl.ds(r0, CRE)], st0)
            pltpu.sync_copy(dis.at[pl.ds(r0, CRE)], dbuf)

            @pl.loop(0, CRE // 16)
            def _(g):
                dvec = dbuf[pl.ds(g * 16, 16)]
                for j in range(16):
                    r = g * 16 + j
                    dv = dvec[j]
                    for hcol in range(2):
                        sl = pl.ds(hcol * 16, 16)
                        x1v = dv * rbuf[0, r, sl]
                        rbuf[1, r, sl] = x1v
                        rbuf[0, r, sl] = dv * x1v

            pltpu.sync_copy(st1, zo.at[pl.ds(r0, CRE)])
            pltpu.sync_copy(st0, acc.at[pl.ds(r0, CRE)])
            pltpu.sync_copy(st0, y1t.at[pl.ds(r0, CRE)])

        plsc.subcore_barrier()
        edge_pass(y1t)
        plsc.subcore_barrier()

        @pl.loop(0, RPT // CRE)
        def _(kc):
            r0 = s * RPT + kc * CRE
            pltpu.sync_copy(acc.at[pl.ds(r0, CRE)], st0)
            pltpu.sync_copy(zo.at[pl.ds(r0, CRE)], st1)
            pltpu.sync_copy(dis.at[pl.ds(r0, CRE)], dbuf)

            @pl.loop(0, CRE // 16)
            def _(g):
                dvec = dbuf[pl.ds(g * 16, 16)]
                for j in range(16):
                    r = g * 16 + j
                    dv = dvec[j]
                    for hcol in range(2):
                        sl = pl.ds(hcol * 16, 16)
                        rbuf[1, r, sl] = (rbuf[1, r, sl] + dv * rbuf[0, r, sl])

            pltpu.sync_copy(x0h.at[pl.ds(r0, CRE)], st0)

            @pl.loop(0, CRE // 16)
            def _(g):
                for j in range(16):
                    r = g * 16 + j
                    for hcol in range(2):
                        sl = pl.ds(hcol * 16, 16)
                        rbuf[1, r, sl] = (rbuf[1, r, sl] + rbuf[0, r, sl]) * (1.0 / 3.0)

            pltpu.sync_copy(st1, zo.at[pl.ds(r0, CRE)])

    @pl.when(c == 0)
    def _():
        half(y_lo, x0_lo, y1_lo, z_lo)

    @pl.when(c == 1)
    def _():
        half(y_hi, x0_hi, y1_hi, z_hi)


_layers = pl.kernel(
    _layers_body,
    out_type=[jax.ShapeDtypeStruct((NP, HD), jnp.float32)] * 4,
    mesh=_mesh(),
    compiler_params=_sc_params,
    scratch_types=[
        pltpu.VMEM_SHARED((NP, HD), jnp.float32),
        pltpu.VMEM((2, KB * 128, HD), jnp.float32),
        pltpu.VMEM((3, KB * 128), jnp.int32),
        pltpu.VMEM((3, KB * 128), jnp.int32),
        pltpu.VMEM((CRE,), jnp.float32),
        pltpu.SemaphoreType.DMA,
        pltpu.SemaphoreType.DMA,
        pltpu.SemaphoreType.DMA,
        pltpu.SemaphoreType.DMA,
        pltpu.SemaphoreType.DMA,
        pltpu.SemaphoreType.DMA,
    ],
)


def _gather_body(z_lo, z_hi, idx1d, out_lo, out_hi,
                 gidx, gbuf, gsem0, gsem1, osem0, osem1):
    c = lax.axis_index("c")
    s = lax.axis_index("s")
    gsems = (gsem0, gsem1)
    osems = (osem0, osem1)
    NSLAB = GPT // 8

    def half(z, out):
        base = s * GPT * 128
        pltpu.sync_copy(idx1d.at[pl.ds(base, GPT * 128)], gidx)

        def fire_g(b, it):
            pltpu.async_copy(z.at[gidx.at[pl.ds(it * 1024, 1024)]],
                             gbuf.at[b], gsems[b])

        def wait_g(b, it):
            pltpu.make_async_copy(z.at[gidx.at[pl.ds(it * 1024, 1024)]],
                                  gbuf.at[b], gsems[b]).wait()

        def fire_o(b, it):
            pltpu.async_copy(gbuf.at[b],
                             out.at[pl.ds(base + it * 1024, 1024)], osems[b])

        def wait_o(b, it):
            pltpu.make_async_copy(gbuf.at[b],
                                  out.at[pl.ds(base + it * 1024, 1024)],
                                  osems[b]).wait()

        fire_g(0, 0)
        for it in range(NSLAB):
            b = it % 2
            wait_g(b, it)
            if it + 1 < NSLAB:
                if it >= 1:
                    wait_o(1 - b, it - 1)
                fire_g(1 - b, it + 1)
            fire_o(b, it)
        wait_o(1, NSLAB - 2)
        wait_o(0, NSLAB - 1)

    @pl.when(c == 0)
    def _():
        half(z_lo, out_lo)

    @pl.when(c == 1)
    def _():
        half(z_hi, out_hi)


_gather = pl.kernel(
    _gather_body,
    out_type=[jax.ShapeDtypeStruct((GIP, HD), jnp.float32)] * 2,
    mesh=_mesh(),
    compiler_params=_sc_params,
    scratch_types=[
        pltpu.VMEM((GPT * 128,), jnp.int32),
        pltpu.VMEM((2, 1024, HD), jnp.float32),
        pltpu.SemaphoreType.DMA,
        pltpu.SemaphoreType.DMA,
        pltpu.SemaphoreType.DMA,
        pltpu.SemaphoreType.DMA,
    ],
)


BS = 512


def _loss_body(ulo, uhi, vlo, vhi, nlo, nhi, w, out):
    i = pl.program_id(0)
    ul = ulo[...]
    uh = uhi[...]
    pos = jnp.sum(ul * vlo[...], axis=1) + jnp.sum(uh * vhi[...], axis=1)
    nl = nlo[...]
    nh = nhi[...]
    neg = jnp.sum(nl * ul[:, None, :], axis=2) + jnp.sum(nh * uh[:, None, :], axis=2)
    coef = -jnp.sign(w[...]) + 2.0
    t = coef[:, None] * pos[:, None] - neg
    ls = jnp.minimum(t, 0.0) - jnp.log1p(jnp.exp(-jnp.abs(t)))
    reg = (jnp.sum(ul * ul) + jnp.sum(uh * uh)
           + jnp.sum(vlo[...] ** 2) + jnp.sum(vhi[...] ** 2)
           + jnp.sum(nl * nl) + jnp.sum(nh * nh))
    val = -jnp.sum(ls) + REG * reg

    @pl.when(i == 0)
    def _():
        out[0, 0] = 0.0

    out[0, 0] += val


_loss = pl.pallas_call(
    _loss_body,
    grid=(B // BS,),
    in_specs=[
        pl.BlockSpec((BS, HD), lambda i: (i, 0)),
        pl.BlockSpec((BS, HD), lambda i: (i, 0)),
        pl.BlockSpec((BS, HD), lambda i: (i, 0)),
        pl.BlockSpec((BS, HD), lambda i: (i, 0)),
        pl.BlockSpec((BS, NNEG, HD), lambda i: (i, 0, 0)),
        pl.BlockSpec((BS, NNEG, HD), lambda i: (i, 0, 0)),
        pl.BlockSpec((BS,), lambda i: (i,)),
    ],
    out_specs=pl.BlockSpec((1, 1), lambda i: (0, 0), memory_space=pltpu.SMEM),
    out_shape=jax.ShapeDtypeStruct((1, 1), jnp.float32),
)


def kernel(E_pos, E_item, u, v, w, n, edge_index):
    x0 = jnp.concatenate([E_pos, E_item], axis=0)
    x0p = jnp.pad(x0, ((0, NP - NN), (0, 0)))
    pad = jnp.full((EP - E,), DUMMY, jnp.int32)
    row2d = jnp.concatenate([edge_index[0], pad]).reshape(ER, 128)
    col2d = jnp.concatenate([edge_index[1], pad]).reshape(ER, 128)

    deg0, deg1 = _deg_kernel(col2d)
    dis, y_lo, y_hi = _prep(deg0, deg1, x0p)

    zin_lo = x0p[:, :HD]
    zin_hi = x0p[:, HD:]
    row1d = row2d.reshape(EP)
    col1d = col2d.reshape(EP)
    _y1lo, _y1hi, zlo, zhi = _layers(y_lo, y_hi, row1d, col1d, dis,
                                     zin_lo, zin_hi)

    gpad = jnp.full((GIP - GI,), DUMMY, jnp.int32)
    all_idx = jnp.concatenate([u, v, n.reshape(-1), gpad])
    rows_lo, rows_hi = _gather(zlo, zhi, all_idx)

    ulo, vlo_r, nlo = rows_lo[:B], rows_lo[B:2 * B], rows_lo[2 * B:GI]
    uhi, vhi_r, nhi = rows_hi[:B], rows_hi[B:2 * B], rows_hi[2 * B:GI]
    out = _loss(ulo, uhi, vlo_r, vhi_r,
                nlo.reshape(B, NNEG, HD), nhi.reshape(B, NNEG, HD), w)
    return out[0, 0]


# speedup vs baseline: 1.8016x; 1.1420x over previous
import functools

import jax
import jax.numpy as jnp
from jax import lax
from jax.experimental import pallas as pl
from jax.experimental.pallas import tpu as pltpu
from jax.experimental.pallas import tpu_sc as plsc

NN = 50000
D = 64
HD = 32
E = 800000
B = 4096
NNEG = 40
REG = 1e-4

NP = 50176
EP = 811008
ER = EP // 128
DUMMY = NN

NC, NS = 2, 16
RPT = NP // NS
CR = 112
KB = 3
EPTR = ER // NS
NIT = EPTR // KB

GI = 2 * B + B * NNEG
GRP = 1408
GIP = GRP * 128
GPT = GRP // NS
GB = 8

_mesh = lambda: plsc.VectorSubcoreMesh(core_axis_name="c", subcore_axis_name="s")
_sc_params = pltpu.CompilerParams(use_tc_tiling_on_sc=False)


def _deg_body(col384, deg0, deg1, deg_sp, onesbuf, idxbuf, zbuf):
    c = lax.axis_index("c")
    s = lax.axis_index("s")
    lane = lax.iota(jnp.int32, 16)
    onev = jnp.where(lane == 0, 1.0, 0.0).astype(jnp.float32)
    zv = jnp.zeros((16,), jnp.float32)

    @pl.loop(0, 384)
    def _(r):
        onesbuf[r] = onev

    @pl.loop(0, CR)
    def _(r):
        zbuf[r] = zv

    @pl.loop(0, RPT // CR)
    def _(k):
        pltpu.sync_copy(zbuf, deg_sp.at[pl.ds(s * RPT + k * CR, CR)])

    plsc.subcore_barrier()

    base = (c * NS + s) * (ER * 128 // 384 // 32)

    @pl.loop(0, 11)
    def _(kc):
        pltpu.sync_copy(col384.at[pl.ds(base + kc * 6, 6)], idxbuf)

        @pl.loop(0, 6)
        def _(j):
            pltpu.sync_copy(onesbuf, deg_sp.at[idxbuf.at[j]], add=True)

    plsc.subcore_barrier()

    @pl.when(c == 0)
    def _():
        @pl.loop(0, RPT // CR)
        def _(k):
            r0 = s * RPT + k * CR
            pltpu.sync_copy(deg_sp.at[pl.ds(r0, CR)], deg0.at[pl.ds(r0, CR)])

    @pl.when(c == 1)
    def _():
        @pl.loop(0, RPT // CR)
        def _(k):
            r0 = s * RPT + k * CR
            pltpu.sync_copy(deg_sp.at[pl.ds(r0, CR)], deg1.at[pl.ds(r0, CR)])


_deg_kernel = pl.kernel(
    _deg_body,
    out_type=[jax.ShapeDtypeStruct((NP, 16), jnp.float32)] * 2,
    mesh=_mesh(),
    compiler_params=_sc_params,
    scratch_types=[
        pltpu.VMEM_SHARED((NP, 16), jnp.float32),
        pltpu.VMEM((384, 16), jnp.float32),
        pltpu.VMEM((6, 384), jnp.int32),
        pltpu.VMEM((CR, 16), jnp.float32),
    ],
)


BN2 = 3136


def _prep_body(dp0, dp1, x0, dis, ylo, yhi):
    i = pl.program_id(0)
    deg = dp0[...][:, 0] + dp1[...][:, 0]
    d = jnp.where(deg > 0, lax.rsqrt(deg), 0.0)
    dis[pl.ds(pl.multiple_of(i * BN2, 128), BN2)] = d
    y = x0[...] * d[:, None]
    ylo[...] = y[:, :HD]
    yhi[...] = y[:, HD:]


_prep = pl.pallas_call(
    _prep_body,
    grid=(NP // BN2,),
    in_specs=[
        pl.BlockSpec((BN2, 16), lambda i: (i, 0)),
        pl.BlockSpec((BN2, 16), lambda i: (i, 0)),
        pl.BlockSpec((BN2, D), lambda i: (i, 0)),
    ],
    out_specs=[
        pl.BlockSpec((NP,), lambda i: (0,)),
        pl.BlockSpec((BN2, HD), lambda i: (i, 0)),
        pl.BlockSpec((BN2, HD), lambda i: (i, 0)),
    ],
    out_shape=[
        jax.ShapeDtypeStruct((NP,), jnp.float32),
        jax.ShapeDtypeStruct((NP, HD), jnp.float32),
        jax.ShapeDtypeStruct((NP, HD), jnp.float32),
    ],
)


CRE = 112


def _layers_body(y_lo, y_hi, row1d, col1d, dis, x0_lo, x0_hi,
                 y1_lo, y1_hi, z_lo, z_hi,
                 acc, rbuf, ridx, cidx, dbuf,
                 gsem0, gsem1, ssem0, ssem1, isem0, isem1):
    c = lax.axis_index("c")
    s = lax.axis_index("s")
    gsems = (gsem0, gsem1)
    ssems = (ssem0, ssem1)
    isems = (isem0, isem1)

    def edge_pass(y):
        base = s * EPTR

        def copy_idx(i, ib, sem):
            e0 = (base + i * KB) * 128
            pltpu.async_copy(row1d.at[pl.ds(e0, KB * 128)], ridx.at[ib], sem)
            pltpu.async_copy(col1d.at[pl.ds(e0, KB * 128)], cidx.at[ib], sem)

        def wait_idx(ib, sem):
            pltpu.make_async_copy(
                row1d.at[pl.ds(0, KB * 128)], ridx.at[ib], sem).wait()
            pltpu.make_async_copy(
                col1d.at[pl.ds(0, KB * 128)], cidx.at[ib], sem).wait()

        def fire_gathers(b, ib):
            pltpu.async_copy(y.at[ridx.at[ib]], rbuf.at[b], gsems[b])

        def fire_scatters(b, ib):
            pltpu.async_copy(rbuf.at[b], acc.at[cidx.at[ib]], ssems[b],
                             add=True)

        def wait_gathers(b, ib):
            pltpu.make_async_copy(y.at[ridx.at[ib]], rbuf.at[b],
                                  gsems[b]).wait()

        def wait_scatters(b, ib):
            pltpu.make_async_copy(
                rbuf.at[b], acc.at[cidx.at[ib]], ssems[b]).wait()

        copy_idx(0, 0, isems[0])
        wait_idx(0, isems[0])
        copy_idx(1, 1, isems[1])
        fire_gathers(0, 0)

        @pl.loop(0, NIT, step=6)
        def _(it):
            for k in range(6):
                i = it + k
                b = k % 2
                ib = k % 3
                wait_gathers(b, ib)

                @pl.when(i + 1 < NIT)
                def _():
                    @pl.when(i >= 1)
                    def _():
                        wait_scatters(1 - b, (k + 2) % 3)

                    @pl.when(i + 2 < NIT)
                    def _():
                        copy_idx(i + 2, (k + 2) % 3, isems[b])

                    wait_idx((k + 1) % 3, isems[1 - b])
                    fire_gathers(1 - b, (k + 1) % 3)

                fire_scatters(b, ib)

        wait_scatters(0, (NIT - 2) % 3)
        wait_scatters(1, (NIT - 1) % 3)

    def half(y0, x0h, y1t, zo):
        st0 = rbuf.at[0, pl.ds(0, CRE)]
        st1 = rbuf.at[1, pl.ds(0, CRE)]

        @pl.loop(0, RPT // CR)
        def _(k):
            r0 = s * RPT + k * CR
            pltpu.sync_copy(y0.at[pl.ds(r0, CR)], acc.at[pl.ds(r0, CR)])

        plsc.subcore_barrier()
        edge_pass(y0)
        plsc.subcore_barrier()

        @pl.loop(0, RPT // CRE)
        def _(kc):
            r0 = s * RPT + kc * CRE
            pltpu.sync_copy(acc.at[pl.ds(r0, CRE)], st0)
            pltpu.sync_copy(dis.at[pl.ds(r0, CRE)], dbuf)

            @pl.loop(0, CRE // 16)
            def _(g):
                dvec = dbuf[pl.ds(g * 16, 16)]
                for j in range(16):
                    r = g * 16 + j
                    dv = dvec[j]
                    for hcol in range(2):
                        sl = pl.ds(hcol * 16, 16)
                        x1v = dv * rbuf[0, r, sl]
                        rbuf[1, r, sl] = x1v
                        rbuf[0, r, sl] = dv * x1v

            pltpu.sync_copy(st1, zo.at[pl.ds(r0, CRE)])
            pltpu.sync_copy(st0, acc.at[pl.ds(r0, CRE)])
            pltpu.sync_copy(st0, y1t.at[pl.ds(r0, CRE)])

        plsc.subcore_barrier()
        edge_pass(y1t)
        plsc.subcore_barrier()

        @pl.loop(0, RPT // CRE)
        def _(kc):
            r0 = s * RPT + kc * CRE
            pltpu.sync_copy(acc.at[pl.ds(r0, CRE)], st0)
            pltpu.sync_copy(zo.at[pl.ds(r0, CRE)], st1)
            pltpu.sync_copy(dis.at[pl.ds(r0, CRE)], dbuf)

            @pl.loop(0, CRE // 16)
            def _(g):
                dvec = dbuf[pl.ds(g * 16, 16)]
                for j in range(16):
                    r = g * 16 + j
                    dv = dvec[j]
                    for hcol in range(2):
                        sl = pl.ds(hcol * 16, 16)
                        rbuf[1, r, sl] = (rbuf[1, r, sl] + dv * rbuf[0, r, sl])

            pltpu.sync_copy(x0h.at[pl.ds(r0, CRE)], st0)

            @pl.loop(0, CRE // 16)
            def _(g):
                for j in range(16):
                    r = g * 16 + j
                    for hcol in range(2):
                        sl = pl.ds(hcol * 16, 16)
                        rbuf[1, r, sl] = (rbuf[1, r, sl] + rbuf[0, r, sl]) * (1.0 / 3.0)

            pltpu.sync_copy(st1, zo.at[pl.ds(r0, CRE)])

    @pl.when(c == 0)
    def _():
        half(y_lo, x0_lo, y1_lo, z_lo)

    @pl.when(c == 1)
    def _():
        half(y_hi, x0_hi, y1_hi, z_hi)


_layers = pl.kernel(
    _layers_body,
    out_type=[jax.ShapeDtypeStruct((NP, HD), jnp.float32)] * 4,
    mesh=_mesh(),
    compiler_params=_sc_params,
    scratch_types=[
        pltpu.VMEM_SHARED((NP, HD), jnp.float32),
        pltpu.VMEM((2, KB * 128, HD), jnp.float32),
        pltpu.VMEM((3, KB * 128), jnp.int32),
        pltpu.VMEM((3, KB * 128), jnp.int32),
        pltpu.VMEM((CRE,), jnp.float32),
        pltpu.SemaphoreType.DMA,
        pltpu.SemaphoreType.DMA,
        pltpu.SemaphoreType.DMA,
        pltpu.SemaphoreType.DMA,
        pltpu.SemaphoreType.DMA,
        pltpu.SemaphoreType.DMA,
    ],
)


def _gather_body(z_lo, z_hi, idx1d, out_lo, out_hi,
                 gidx, gbuf, gsem0, gsem1, osem0, osem1):
    c = lax.axis_index("c")
    s = lax.axis_index("s")
    gsems = (gsem0, gsem1)
    osems = (osem0, osem1)
    NSLAB = GPT // 8

    def half(z, out):
        base = s * GPT * 128
        pltpu.sync_copy(idx1d.at[pl.ds(base, GPT * 128)], gidx)

        def fire_g(b, it):
            pltpu.async_copy(z.at[gidx.at[pl.ds(it * 1024, 1024)]],
                             gbuf.at[b], gsems[b])

        def wait_g(b, it):
            pltpu.make_async_copy(z.at[gidx.at[pl.ds(it * 1024, 1024)]],
                                  gbuf.at[b], gsems[b]).wait()

        def fire_o(b, it):
            pltpu.async_copy(gbuf.at[b],
                             out.at[pl.ds(base + it * 1024, 1024)], osems[b])

        def wait_o(b, it):
            pltpu.make_async_copy(gbuf.at[b],
                                  out.at[pl.ds(base + it * 1024, 1024)],
                                  osems[b]).wait()

        fire_g(0, 0)
        for it in range(NSLAB):
            b = it % 2
            wait_g(b, it)
            if it + 1 < NSLAB:
                if it >= 1:
                    wait_o(1 - b, it - 1)
                fire_g(1 - b, it + 1)
            fire_o(b, it)
        wait_o(1, NSLAB - 2)
        wait_o(0, NSLAB - 1)

    @pl.when(c == 0)
    def _():
        half(z_lo, out_lo)

    @pl.when(c == 1)
    def _():
        half(z_hi, out_hi)


_gather = pl.kernel(
    _gather_body,
    out_type=[jax.ShapeDtypeStruct((GIP, HD), jnp.float32)] * 2,
    mesh=_mesh(),
    compiler_params=_sc_params,
    scratch_types=[
        pltpu.VMEM((GPT * 128,), jnp.int32),
        pltpu.VMEM((2, 1024, HD), jnp.float32),
        pltpu.SemaphoreType.DMA,
        pltpu.SemaphoreType.DMA,
        pltpu.SemaphoreType.DMA,
        pltpu.SemaphoreType.DMA,
    ],
)


BS = 512


def _loss_body(nlo, ulo, vlo, nhi, uhi, vhi, w, out):
    i = pl.program_id(0)
    ul = ulo[...]
    uh = uhi[...]
    pos = jnp.sum(ul * vlo[...], axis=1) + jnp.sum(uh * vhi[...], axis=1)
    nl = nlo[...].reshape(BS, NNEG, HD)
    nh = nhi[...].reshape(BS, NNEG, HD)
    neg = jnp.sum(nl * ul[:, None, :], axis=2) + jnp.sum(nh * uh[:, None, :], axis=2)
    coef = -jnp.sign(w[...]) + 2.0
    t = coef[:, None] * pos[:, None] - neg
    ls = jnp.minimum(t, 0.0) - jnp.log1p(jnp.exp(-jnp.abs(t)))
    reg = (jnp.sum(ul * ul) + jnp.sum(uh * uh)
           + jnp.sum(vlo[...] ** 2) + jnp.sum(vhi[...] ** 2)
           + jnp.sum(nl * nl) + jnp.sum(nh * nh))
    val = -jnp.sum(ls) + REG * reg

    @pl.when(i == 0)
    def _():
        out[0, 0] = 0.0

    out[0, 0] += val


_loss = pl.pallas_call(
    _loss_body,
    grid=(B // BS,),
    in_specs=[
        pl.BlockSpec((BS * NNEG, HD), lambda i: (i, 0)),
        pl.BlockSpec((BS, HD), lambda i: (i + (B * NNEG) // BS, 0)),
        pl.BlockSpec((BS, HD), lambda i: (i + (B * NNEG + B) // BS, 0)),
        pl.BlockSpec((BS * NNEG, HD), lambda i: (i, 0)),
        pl.BlockSpec((BS, HD), lambda i: (i + (B * NNEG) // BS, 0)),
        pl.BlockSpec((BS, HD), lambda i: (i + (B * NNEG + B) // BS, 0)),
        pl.BlockSpec((BS,), lambda i: (i,)),
    ],
    out_specs=pl.BlockSpec((1, 1), lambda i: (0, 0), memory_space=pltpu.SMEM),
    out_shape=jax.ShapeDtypeStruct((1, 1), jnp.float32),
)


def kernel(E_pos, E_item, u, v, w, n, edge_index):
    x0 = jnp.concatenate([E_pos, E_item], axis=0)
    x0p = jnp.pad(x0, ((0, NP - NN), (0, 0)))
    pad = jnp.full((EP - E,), DUMMY, jnp.int32)
    edge_row = jnp.concatenate([edge_index[0], pad])
    edge_col = jnp.concatenate([edge_index[1], pad])

    col384 = edge_col.reshape(EP // 384, 384)
    deg0, deg1 = _deg_kernel(col384)
    dis, y_lo, y_hi = _prep(deg0, deg1, x0p)

    zin_lo = x0p[:, :HD]
    zin_hi = x0p[:, HD:]
    row1d = edge_row
    col1d = edge_col
    _y1lo, _y1hi, zlo, zhi = _layers(y_lo, y_hi, row1d, col1d, dis,
                                     zin_lo, zin_hi)

    gpad = jnp.full((GIP - GI,), DUMMY, jnp.int32)
    all_idx = jnp.concatenate([n.reshape(-1), u, v, gpad])
    rows_lo, rows_hi = _gather(zlo, zhi, all_idx)

    out = _loss(rows_lo, rows_lo, rows_lo, rows_hi, rows_hi, rows_hi, w)
    return out[0, 0]
